# baseline (speedup 1.0000x reference)
import functools

import jax
import jax.numpy as jnp
from jax import lax
from jax.experimental import pallas as pl
from jax.experimental.pallas import tpu as pltpu
from jax.experimental.pallas import tpu_sc as plsc

N0, N1, N2, C = 10000, 320000, 100000, 128

NC, NS, L = 2, 16, 16
GB = 128
IDXBUF = 1024
ZB = 32


def _mm_body(ns_, x_ref, *refs):
    w_refs = refs[:ns_]
    o_refs = refs[ns_:]
    x = x_ref[...]
    for w_ref, o_ref in zip(w_refs, o_refs):
        o_ref[...] = jnp.dot(x, w_ref[...],
                             preferred_element_type=jnp.float32)


def _matmuls(x, ws):
    n, c = x.shape
    blk = 2000
    assert n % blk == 0
    ns_ = len(ws)
    wspec = pl.BlockSpec((c, c), lambda i: (0, 0))
    outs = pl.pallas_call(
        functools.partial(_mm_body, ns_),
        grid=(n // blk,),
        in_specs=[pl.BlockSpec((blk, c), lambda i: (i, 0))] + [wspec] * ns_,
        out_specs=[pl.BlockSpec((blk, c), lambda i: (i, 0))] * ns_,
        out_shape=[jax.ShapeDtypeStruct((n, c), jnp.float32)] * ns_,
    )(x, *ws)
    return outs


def _make_sc_spmm(n_out, r_chunk, npasses, nnzs):
    ns = len(nnzs)
    R = r_chunk
    assert R % 128 == 0
    AR = R + 128
    npad = 2 * R * npasses
    zpt = AR // NS
    dpt = R // NS
    i32 = jnp.int32

    STG = IDXBUF + GB + L

    def body(*refs):
        ins = refs[:4 * ns]
        out = refs[4 * ns]
        (acc, rbuf0, rbuf1, cbuf0, cbuf1, vbuf0, vbuf1,
         sg_col, sg_lrow, sg_val, gidxb, lidxb,
         gbuf, zbuf, smg0, smg1, sms0, sms1, smi0, smi1) = refs[4 * ns + 1:]
        rbufs, cbufs, vbufs = (rbuf0, rbuf1), (cbuf0, cbuf1), (vbuf0, vbuf1)
        cid = lax.axis_index("c")
        sid = lax.axis_index("s")
        iota = lax.iota(i32, L)
        zero16 = jnp.zeros((L,), jnp.float32)

        def _zrow(j, _):
            for k in range(C // L):
                zbuf[j, pl.ds(k * L, L)] = zero16
            return 0
        lax.fori_loop(0, ZB, _zrow, 0)

        def one_pass(p, _):
            base = (2 * p + cid) * R
            zrow0 = sid * zpt
            for i in range(zpt // ZB):
                pltpu.sync_copy(zbuf, acc.at[pl.ds(zrow0 + i * ZB, ZB)])
            if zpt % ZB:
                pltpu.sync_copy(zbuf.at[pl.ds(0, zpt % ZB)],
                                acc.at[pl.ds(zrow0 + (zpt // ZB) * ZB,
                                             zpt % ZB)])
            plsc.subcore_barrier()

            for s in range(ns):
                rows_h, cols_h, vals_h, src_h = ins[4 * s:4 * s + 4]
                b16 = nnzs[s] // L
                base_blk, rem = b16 // NS, b16 % NS
                my_start = (sid * base_blk + jnp.minimum(sid, rem)) * L
                count = (base_blk + jnp.where(sid < rem, 1, 0)) * L
                nchunks = (count + IDXBUF - 1) // IDXBUF

                def make_step(sl):
                    def step(i, cur):
                        r = rbufs[sl][pl.ds(i * L, L)]
                        rl = r - base
                        m = (rl >= 0) & (rl < R)

                        def append(cu):
                            cnt = jnp.sum(jnp.where(m, jnp.int32(1),
                                                    jnp.int32(0)))
                            plsc.store_compressed(sg_col.at[pl.ds(cu, L)],
                                                  cbufs[sl][pl.ds(i * L, L)],
                                                  mask=m)
                            plsc.store_compressed(sg_lrow.at[pl.ds(cu, L)],
                                                  rl, mask=m)
                            plsc.store_compressed(sg_val.at[pl.ds(cu, L)],
                                                  vbufs[sl][pl.ds(i * L, L)],
                                                  mask=m)
                            return cu + cnt

                        return lax.cond(jnp.any(m), append,
                                        lambda cu: cu, cur)
                    return step

                def issue_idx(k, sl, sem):
                    off = jnp.minimum(k * IDXBUF, count - IDXBUF)
                    pltpu.async_copy(
                        rows_h.at[pl.ds(my_start + off, IDXBUF)],
                        rbufs[sl], sem)
                    pltpu.async_copy(
                        cols_h.at[pl.ds(my_start + off, IDXBUF)],
                        cbufs[sl], sem)
                    pltpu.async_copy(
                        vals_h.at[pl.ds(my_start + off, IDXBUF)],
                        vbufs[sl], sem)

                def wait_idx(sl, sem):
                    pltpu.make_async_copy(rows_h.at[pl.ds(0, IDXBUF)],
                                          rbufs[sl], sem).wait()
                    pltpu.make_async_copy(cols_h.at[pl.ds(0, IDXBUF)],
                                          cbufs[sl], sem).wait()
                    pltpu.make_async_copy(vals_h.at[pl.ds(0, IDXBUF)],
                                          vbufs[sl], sem).wait()

                def issue_gather(k2, slot, sem):
                    for t in range(GB // L):
                        gidxb[slot, pl.ds(t * L, L)] = \
                            sg_col[pl.ds(k2 * GB + t * L, L)]
                    pltpu.async_copy(src_h.at[gidxb.at[slot]],
                                     gbuf.at[slot], sem)

                def issue_scatter(k2, slot, sem):
                    for t in range(GB // L):
                        lidxb[slot, pl.ds(t * L, L)] = \
                            sg_lrow[pl.ds(k2 * GB + t * L, L)]
                    pltpu.async_copy(gbuf.at[slot], acc.at[lidxb.at[slot]],
                                     sem, add=True)

                def wait_gather(slot, sem):
                    pltpu.make_async_copy(src_h.at[gidxb.at[slot]],
                                          gbuf.at[slot], sem).wait()

                def wait_scatter(slot, sem):
                    pltpu.make_async_copy(gbuf.at[slot],
                                          acc.at[lidxb.at[slot]],
                                          sem).wait()

                def with_slot(slot, fn):
                    @pl.when(slot == 0)
                    def _a():
                        fn(0, smg0, sms0)

                    @pl.when(slot == 1)
                    def _b():
                        fn(1, smg1, sms1)

                def scale(k2, slot):
                    def srow(j, _):
                        spl = plsc.load_gather(
                            sg_val, [jnp.broadcast_to(k2 * GB + j, (L,))])
                        for t in range(C // L):
                            gbuf[slot, j, pl.ds(t * L, L)] = \
                                gbuf[slot, j, pl.ds(t * L, L)] * spl
                        return 0
                    lax.fori_loop(0, GB, srow, 0)

                def flush_batches(nb):
                    @pl.when(nb > 0)
                    def _flush():
                        with_slot(0, lambda sl, sg, ss: issue_gather(0, sl, sg))

                        def fl(k2, _):
                            slot = k2 & 1

                            def work(sl, sg, ss):
                                wait_gather(sl, sg)
                                scale(k2, sl)
                                issue_scatter(k2, sl, ss)

                                @pl.when((k2 + 1 < nb) & (k2 >= 1))
                                def _pre():
                                    osl = 1 - sl
                                    osg, oss = (smg1, sms1) if sl == 0 \
                                        else (smg0, sms0)
                                    wait_scatter(osl, oss)
                                    issue_gather(k2 + 1, osl, osg)

                                @pl.when((k2 + 1 < nb) & (k2 == 0))
                                def _pre0():
                                    issue_gather(k2 + 1, 1 - sl,
                                                 smg1 if sl == 0 else smg0)

                            with_slot(slot, work)
                            return 0

                        lax.fori_loop(0, nb, fl, 0)
                        with_slot((nb - 1) & 1,
                                  lambda sl, sg, ss: wait_scatter(sl, ss))

                        @pl.when(nb >= 2)
                        def _last2():
                            with_slot(nb & 1,
                                      lambda sl, sg, ss: wait_scatter(sl, ss))

                def chunk(k, cin):
                    off = jnp.minimum(k * IDXBUF, count - IDXBUF)
                    lo = jnp.maximum(0, k * IDXBUF - off) // L

                    @pl.when(k + 1 < nchunks)
                    def _prefetch():
                        @pl.when((k & 1) == 0)
                        def _p0():
                            issue_idx(k + 1, 1, smi1)

                        @pl.when((k & 1) == 1)
                        def _p1():
                            issue_idx(k + 1, 0, smi0)

                    def scan_with(sl, sem):
                        def go(c0):
                            wait_idx(sl, sem)
                            return lax.fori_loop(lo, IDXBUF // L,
                                                 make_step(sl), c0)
                        return go

                    cur = lax.cond((k & 1) == 0,
                                   scan_with(0, smi0), scan_with(1, smi1),
                                   cin)
                    nfull = cur // GB
                    flush_batches(nfull)
                    rem_off = nfull * GB
                    for t in range(GB // L):
                        sg_col[pl.ds(t * L, L)] = \
                            sg_col[pl.ds(rem_off + t * L, L)]
                        sg_lrow[pl.ds(t * L, L)] = \
                            sg_lrow[pl.ds(rem_off + t * L, L)]
                        sg_val[pl.ds(t * L, L)] = \
                            sg_val[pl.ds(rem_off + t * L, L)]
                    return cur - rem_off

                issue_idx(0, 0, smi0)
                cur_f = lax.fori_loop(0, nchunks, chunk, jnp.int32(0))
                for t in range(GB // L):
                    st = cur_f + t * L
                    sg_col[pl.ds(st, L)] = iota
                    sg_lrow[pl.ds(st, L)] = R + ((t * L + iota) & (GB - 1))
                    sg_val[pl.ds(st, L)] = zero16
                flush_batches(jnp.where(cur_f > 0, 1, 0))

            plsc.subcore_barrier()
            orow0 = base + sid * dpt
            drow0 = sid * dpt
            for i in range(dpt // ZB):
                pltpu.sync_copy(acc.at[pl.ds(drow0 + i * ZB, ZB)],
                                out.at[pl.ds(orow0 + i * ZB, ZB)])
            if dpt % ZB:
                pltpu.sync_copy(
                    acc.at[pl.ds(drow0 + (dpt // ZB) * ZB, dpt % ZB)],
                    out.at[pl.ds(orow0 + (dpt // ZB) * ZB, dpt % ZB)])
            plsc.subcore_barrier()
            return 0

        lax.fori_loop(0, npasses, one_pass, 0)

    mesh = plsc.VectorSubcoreMesh(core_axis_name="c", subcore_axis_name="s")
    return pl.kernel(
        body,
        out_type=jax.ShapeDtypeStruct((npad, C), jnp.float32),
        mesh=mesh,
        compiler_params=pltpu.CompilerParams(needs_layout_passes=False),
        scratch_types=[
            pltpu.VMEM_SHARED((AR, C), jnp.float32),
            pltpu.VMEM((IDXBUF,), i32),
            pltpu.VMEM((IDXBUF,), i32),
            pltpu.VMEM((IDXBUF,), i32),
            pltpu.VMEM((IDXBUF,), i32),
            pltpu.VMEM((IDXBUF,), jnp.float32),
            pltpu.VMEM((IDXBUF,), jnp.float32),
            pltpu.VMEM((STG,), i32),
            pltpu.VMEM((STG,), i32),
            pltpu.VMEM((STG,), jnp.float32),
            pltpu.VMEM((2, GB), i32),
            pltpu.VMEM((2, GB), i32),
            pltpu.VMEM((2, GB, C), jnp.float32),
            pltpu.VMEM((ZB, C), jnp.float32),
            pltpu.SemaphoreType.DMA,
            pltpu.SemaphoreType.DMA,
            pltpu.SemaphoreType.DMA,
            pltpu.SemaphoreType.DMA,
            pltpu.SemaphoreType.DMA,
            pltpu.SemaphoreType.DMA,
        ],
    )


def _sc_spmm(n_out, r_chunk, streams):
    npasses = -(-n_out // (2 * r_chunk))
    nnzs = tuple(int(s[0].shape[0]) for s in streams)
    k = _make_sc_spmm(n_out, r_chunk, npasses, nnzs)
    flat = []
    for s in streams:
        flat.extend(s)
    outp = k(*flat)
    return outp[:n_out]


def _gn_stats_body(x_ref, sx_ref, sxx_ref):
    @pl.when(pl.program_id(0) == 0)
    def _init():
        sx_ref[...] = jnp.zeros_like(sx_ref)
        sxx_ref[...] = jnp.zeros_like(sxx_ref)

    x = x_ref[...]
    sx_ref[...] += jnp.sum(x, axis=0, keepdims=True)
    sxx_ref[...] += jnp.sum(x * x, axis=0, keepdims=True)


def _gn_apply_body(n, eps, x_ref, sx_ref, sxx_ref, w_ref, b_ref, a_ref, o_ref):
    inv_n = 1.0 / n
    mean = sx_ref[...] * inv_n
    ex2 = sxx_ref[...] * inv_n
    m2 = mean * a_ref[...]
    var = ex2 - 2.0 * m2 * mean + m2 * m2
    std = jnp.sqrt(var + eps)
    scale = w_ref[...] / std
    off = b_ref[...] - m2 * scale
    o_ref[...] = x_ref[...] * scale + off


def _graph_norm(x, weight, bias, mean_scale, eps=1e-5):
    n, c = x.shape
    blk = 2000 if n % 2000 == 0 else 625
    sx, sxx = pl.pallas_call(
        _gn_stats_body,
        grid=(n // blk,),
        in_specs=[pl.BlockSpec((blk, c), lambda i: (i, 0))],
        out_specs=[pl.BlockSpec((1, c), lambda i: (0, 0))] * 2,
        out_shape=[jax.ShapeDtypeStruct((1, c), jnp.float32)] * 2,
    )(x)
    w = weight.reshape(1, c)
    b = bias.reshape(1, c)
    a = mean_scale.reshape(1, c)
    full = pl.BlockSpec((1, c), lambda i: (0, 0))
    return pl.pallas_call(
        functools.partial(_gn_apply_body, float(n), eps),
        grid=(n // blk,),
        in_specs=[pl.BlockSpec((blk, c), lambda i: (i, 0)),
                  full, full, full, full, full],
        out_specs=pl.BlockSpec((blk, c), lambda i: (i, 0)),
        out_shape=jax.ShapeDtypeStruct((n, c), jnp.float32),
    )(x, sx, sxx, w, b, a)


def kernel(x_0, x_1, x_2, L0_rows, L0_cols, L0_vals, L1d_rows, L1d_cols,
           L1d_vals, L1u_rows, L1u_cols, L1u_vals, L2d_rows, L2d_cols,
           L2d_vals, L2u_rows, L2u_cols, L2u_vals, B1_rows, B1_cols, B1_vals,
           B2_rows, B2_cols, B2_vals, batch, batch_1, W0, W01, W1d, W1u, W10,
           W12, W2d, W2u, W21, Wa1, Wa2, gn1_w, gn1_b, gn1_a, gn2_w, gn2_b,
           gn2_a):
    p0, p10 = _matmuls(x_0, [W0, W10])
    p01, p1d, p1u, p21 = _matmuls(x_1, [W01, W1d, W1u, W21])
    p2d, p2u, p12 = _matmuls(x_2, [W2d, W2u, W12])
    x2_out = _sc_spmm(N2, 10240, [
        (L2d_rows, L2d_cols, L2d_vals, p2d),
        (L2u_rows, L2u_cols, L2u_vals, p2u),
        (B2_cols, B2_rows, B2_vals, p21),
    ])
    (pa1,) = _matmuls(x2_out, [Wa1])
    y1 = _sc_spmm(N1, 10240, [
        (L1d_rows, L1d_cols, L1d_vals, p1d),
        (L1u_rows, L1u_cols, L1u_vals, p1u),
        (B1_cols, B1_rows, B1_vals, p10),
        (B2_rows, B2_cols, B2_vals, p12),
        (B2_rows, B2_cols, B2_vals, pa1),
    ])
    x1_out = _graph_norm(y1, gn1_w, gn1_b, gn1_a)
    (pa2,) = _matmuls(x1_out, [Wa2])
    y0 = _sc_spmm(N0, 5120, [
        (L0_rows, L0_cols, L0_vals, p0),
        (B1_rows, B1_cols, B1_vals, p01),
        (B1_rows, B1_cols, B1_vals, pa2),
    ])
    x0_out = _graph_norm(y0, gn2_w, gn2_b, gn2_a)
    return (x0_out, x1_out, x2_out)


# speedup vs baseline: 1.4943x; 1.4943x over previous
import functools

import jax
import jax.numpy as jnp
from jax import lax
from jax.experimental import pallas as pl
from jax.experimental.pallas import tpu as pltpu
from jax.experimental.pallas import tpu_sc as plsc

N0, N1, N2, C = 10000, 320000, 100000, 128

NC, NS, L = 2, 16, 16
GB = 128
IDXBUF = 1024
ZB = 32


def _mm_body(ns_, x_ref, *refs):
    w_refs = refs[:ns_]
    o_refs = refs[ns_:]
    x = x_ref[...]
    for w_ref, o_ref in zip(w_refs, o_refs):
        o_ref[...] = jnp.dot(x, w_ref[...],
                             preferred_element_type=jnp.float32)


def _mm2_body(xa_ref, xb_ref, wa_ref, wb_ref, o_ref):
    o_ref[...] = (jnp.dot(xa_ref[...], wa_ref[...],
                          preferred_element_type=jnp.float32)
                  + jnp.dot(xb_ref[...], wb_ref[...],
                            preferred_element_type=jnp.float32))


def _matmul2(xa, wa, xb, wb):
    n, c = xa.shape
    blk = 2000
    assert n % blk == 0
    wspec = pl.BlockSpec((c, c), lambda i: (0, 0))
    xspec = pl.BlockSpec((blk, c), lambda i: (i, 0))
    return pl.pallas_call(
        _mm2_body,
        grid=(n // blk,),
        in_specs=[xspec, xspec, wspec, wspec],
        out_specs=xspec,
        out_shape=jax.ShapeDtypeStruct((n, c), jnp.float32),
    )(xa, xb, wa, wb)


def _matmuls(x, ws):
    n, c = x.shape
    blk = 2000
    assert n % blk == 0
    ns_ = len(ws)
    wspec = pl.BlockSpec((c, c), lambda i: (0, 0))
    outs = pl.pallas_call(
        functools.partial(_mm_body, ns_),
        grid=(n // blk,),
        in_specs=[pl.BlockSpec((blk, c), lambda i: (i, 0))] + [wspec] * ns_,
        out_specs=[pl.BlockSpec((blk, c), lambda i: (i, 0))] * ns_,
        out_shape=[jax.ShapeDtypeStruct((n, c), jnp.float32)] * ns_,
    )(x, *ws)
    return outs


def _make_sc_spmm(n_out, r_chunk, npasses, nnzs):
    ns = len(nnzs)
    R = r_chunk
    assert R % 128 == 0
    AR = R + 128
    npad = 2 * R * npasses
    zpt = AR // NS
    dpt = R // NS
    i32 = jnp.int32

    STG = IDXBUF + GB + L

    def body(*refs):
        ins = refs[:4 * ns]
        out = refs[4 * ns]
        (acc, rbuf0, rbuf1, cbuf0, cbuf1, vbuf0, vbuf1,
         sg_col, sg_lrow, sg_val, gidxb, lidxb,
         gbuf, zbuf, smg0, smg1, sms0, sms1, smi0, smi1) = refs[4 * ns + 1:]
        rbufs, cbufs, vbufs = (rbuf0, rbuf1), (cbuf0, cbuf1), (vbuf0, vbuf1)
        cid = lax.axis_index("c")
        sid = lax.axis_index("s")
        iota = lax.iota(i32, L)
        zero16 = jnp.zeros((L,), jnp.float32)

        def _zrow(j, _):
            for k in range(C // L):
                zbuf[j, pl.ds(k * L, L)] = zero16
            return 0
        lax.fori_loop(0, ZB, _zrow, 0)

        def one_pass(p, _):
            base = (2 * p + cid) * R
            zrow0 = sid * zpt
            for i in range(zpt // ZB):
                pltpu.sync_copy(zbuf, acc.at[pl.ds(zrow0 + i * ZB, ZB)])
            if zpt % ZB:
                pltpu.sync_copy(zbuf.at[pl.ds(0, zpt % ZB)],
                                acc.at[pl.ds(zrow0 + (zpt // ZB) * ZB,
                                             zpt % ZB)])
            plsc.subcore_barrier()

            for s in range(ns):
                rows_h, cols_h, vals_h, src_h = ins[4 * s:4 * s + 4]
                b16 = nnzs[s] // L
                base_blk, rem = b16 // NS, b16 % NS
                my_start = (sid * base_blk + jnp.minimum(sid, rem)) * L
                count = (base_blk + jnp.where(sid < rem, 1, 0)) * L
                nchunks = (count + IDXBUF - 1) // IDXBUF

                def make_step(sl):
                    def step(i, cur):
                        r = rbufs[sl][pl.ds(i * L, L)]
                        rl = r - base
                        m = (rl >= 0) & (rl < R)
                        cnt = jnp.sum(jnp.where(m, jnp.int32(1),
                                                jnp.int32(0)))
                        plsc.store_compressed(sg_col.at[pl.ds(cur, L)],
                                              cbufs[sl][pl.ds(i * L, L)],
                                              mask=m)
                        plsc.store_compressed(sg_lrow.at[pl.ds(cur, L)],
                                              rl, mask=m)
                        plsc.store_compressed(sg_val.at[pl.ds(cur, L)],
                                              vbufs[sl][pl.ds(i * L, L)],
                                              mask=m)
                        return cur + cnt
                    return step

                def issue_idx(k, sl, sem):
                    off = jnp.minimum(k * IDXBUF, count - IDXBUF)
                    pltpu.async_copy(
                        rows_h.at[pl.ds(my_start + off, IDXBUF)],
                        rbufs[sl], sem)
                    pltpu.async_copy(
                        cols_h.at[pl.ds(my_start + off, IDXBUF)],
                        cbufs[sl], sem)
                    pltpu.async_copy(
                        vals_h.at[pl.ds(my_start + off, IDXBUF)],
                        vbufs[sl], sem)

                def wait_idx(sl, sem):
                    pltpu.make_async_copy(rows_h.at[pl.ds(0, IDXBUF)],
                                          rbufs[sl], sem).wait()
                    pltpu.make_async_copy(cols_h.at[pl.ds(0, IDXBUF)],
                                          cbufs[sl], sem).wait()
                    pltpu.make_async_copy(vals_h.at[pl.ds(0, IDXBUF)],
                                          vbufs[sl], sem).wait()

                def issue_gather(k2, slot, sem):
                    for t in range(GB // L):
                        gidxb[slot, pl.ds(t * L, L)] = \
                            sg_col[pl.ds(k2 * GB + t * L, L)]
                    pltpu.async_copy(src_h.at[gidxb.at[slot]],
                                     gbuf.at[slot], sem)

                def issue_scatter(k2, slot, sem):
                    for t in range(GB // L):
                        lidxb[slot, pl.ds(t * L, L)] = \
                            sg_lrow[pl.ds(k2 * GB + t * L, L)]
                    pltpu.async_copy(gbuf.at[slot], acc.at[lidxb.at[slot]],
                                     sem, add=True)

                def wait_gather(slot, sem):
                    pltpu.make_async_copy(src_h.at[gidxb.at[slot]],
                                          gbuf.at[slot], sem).wait()

                def wait_scatter(slot, sem):
                    pltpu.make_async_copy(gbuf.at[slot],
                                          acc.at[lidxb.at[slot]],
                                          sem).wait()

                def with_slot(slot, fn):
                    @pl.when(slot == 0)
                    def _a():
                        fn(0, smg0, sms0)

                    @pl.when(slot == 1)
                    def _b():
                        fn(1, smg1, sms1)

                def scale(k2, slot):
                    def srow(j, _):
                        spl = plsc.load_gather(
                            sg_val, [jnp.broadcast_to(k2 * GB + j, (L,))])
                        for t in range(C // L):
                            gbuf[slot, j, pl.ds(t * L, L)] = \
                                gbuf[slot, j, pl.ds(t * L, L)] * spl
                        return 0
                    lax.fori_loop(0, GB, srow, 0)

                def flush_batches(nb):
                    @pl.when(nb > 0)
                    def _flush():
                        with_slot(0, lambda sl, sg, ss: issue_gather(0, sl, sg))

                        def fl(k2, _):
                            slot = k2 & 1

                            def work(sl, sg, ss):
                                wait_gather(sl, sg)
                                scale(k2, sl)
                                issue_scatter(k2, sl, ss)

                                @pl.when((k2 + 1 < nb) & (k2 >= 1))
                                def _pre():
                                    osl = 1 - sl
                                    osg, oss = (smg1, sms1) if sl == 0 \
                                        else (smg0, sms0)
                                    wait_scatter(osl, oss)
                                    issue_gather(k2 + 1, osl, osg)

                                @pl.when((k2 + 1 < nb) & (k2 == 0))
                                def _pre0():
                                    issue_gather(k2 + 1, 1 - sl,
                                                 smg1 if sl == 0 else smg0)

                            with_slot(slot, work)
                            return 0

                        lax.fori_loop(0, nb, fl, 0)
                        with_slot((nb - 1) & 1,
                                  lambda sl, sg, ss: wait_scatter(sl, ss))

                        @pl.when(nb >= 2)
                        def _last2():
                            with_slot(nb & 1,
                                      lambda sl, sg, ss: wait_scatter(sl, ss))

                def chunk(k, cin):
                    off = jnp.minimum(k * IDXBUF, count - IDXBUF)
                    lo = jnp.maximum(0, k * IDXBUF - off) // L

                    @pl.when(k + 1 < nchunks)
                    def _prefetch():
                        @pl.when((k & 1) == 0)
                        def _p0():
                            issue_idx(k + 1, 1, smi1)

                        @pl.when((k & 1) == 1)
                        def _p1():
                            issue_idx(k + 1, 0, smi0)

                    def scan_with(sl, sem):
                        def go(c0):
                            wait_idx(sl, sem)
                            return lax.fori_loop(lo, IDXBUF // L,
                                                 make_step(sl), c0)
                        return go

                    cur = lax.cond((k & 1) == 0,
                                   scan_with(0, smi0), scan_with(1, smi1),
                                   cin)
                    nfull = cur // GB
                    flush_batches(nfull)
                    rem_off = nfull * GB
                    for t in range(GB // L):
                        sg_col[pl.ds(t * L, L)] = \
                            sg_col[pl.ds(rem_off + t * L, L)]
                        sg_lrow[pl.ds(t * L, L)] = \
                            sg_lrow[pl.ds(rem_off + t * L, L)]
                        sg_val[pl.ds(t * L, L)] = \
                            sg_val[pl.ds(rem_off + t * L, L)]
                    return cur - rem_off

                issue_idx(0, 0, smi0)
                cur_f = lax.fori_loop(0, nchunks, chunk, jnp.int32(0))
                for t in range(GB // L):
                    st = cur_f + t * L
                    sg_col[pl.ds(st, L)] = iota
                    sg_lrow[pl.ds(st, L)] = R + ((t * L + iota) & (GB - 1))
                    sg_val[pl.ds(st, L)] = zero16
                flush_batches(jnp.where(cur_f > 0, 1, 0))

            plsc.subcore_barrier()
            orow0 = base + sid * dpt
            drow0 = sid * dpt
            for i in range(dpt // ZB):
                pltpu.sync_copy(acc.at[pl.ds(drow0 + i * ZB, ZB)],
                                out.at[pl.ds(orow0 + i * ZB, ZB)])
            if dpt % ZB:
                pltpu.sync_copy(
                    acc.at[pl.ds(drow0 + (dpt // ZB) * ZB, dpt % ZB)],
                    out.at[pl.ds(orow0 + (dpt // ZB) * ZB, dpt % ZB)])
            plsc.subcore_barrier()
            return 0

        lax.fori_loop(0, npasses, one_pass, 0)

    mesh = plsc.VectorSubcoreMesh(core_axis_name="c", subcore_axis_name="s")
    return pl.kernel(
        body,
        out_type=jax.ShapeDtypeStruct((npad, C), jnp.float32),
        mesh=mesh,
        compiler_params=pltpu.CompilerParams(needs_layout_passes=False),
        scratch_types=[
            pltpu.VMEM_SHARED((AR, C), jnp.float32),
            pltpu.VMEM((IDXBUF,), i32),
            pltpu.VMEM((IDXBUF,), i32),
            pltpu.VMEM((IDXBUF,), i32),
            pltpu.VMEM((IDXBUF,), i32),
            pltpu.VMEM((IDXBUF,), jnp.float32),
            pltpu.VMEM((IDXBUF,), jnp.float32),
            pltpu.VMEM((STG,), i32),
            pltpu.VMEM((STG,), i32),
            pltpu.VMEM((STG,), jnp.float32),
            pltpu.VMEM((2, GB), i32),
            pltpu.VMEM((2, GB), i32),
            pltpu.VMEM((2, GB, C), jnp.float32),
            pltpu.VMEM((ZB, C), jnp.float32),
            pltpu.SemaphoreType.DMA,
            pltpu.SemaphoreType.DMA,
            pltpu.SemaphoreType.DMA,
            pltpu.SemaphoreType.DMA,
            pltpu.SemaphoreType.DMA,
            pltpu.SemaphoreType.DMA,
        ],
    )


def _sc_spmm(n_out, r_chunk, streams):
    npasses = -(-n_out // (2 * r_chunk))
    nnzs = tuple(int(s[0].shape[0]) for s in streams)
    k = _make_sc_spmm(n_out, r_chunk, npasses, nnzs)
    flat = []
    for s in streams:
        flat.extend(s)
    outp = k(*flat)
    return outp[:n_out]


def _gn_stats_body(x_ref, sx_ref, sxx_ref):
    @pl.when(pl.program_id(0) == 0)
    def _init():
        sx_ref[...] = jnp.zeros_like(sx_ref)
        sxx_ref[...] = jnp.zeros_like(sxx_ref)

    x = x_ref[...]
    sx_ref[...] += jnp.sum(x, axis=0, keepdims=True)
    sxx_ref[...] += jnp.sum(x * x, axis=0, keepdims=True)


def _gn_apply_body(n, eps, x_ref, sx_ref, sxx_ref, w_ref, b_ref, a_ref, o_ref):
    inv_n = 1.0 / n
    mean = sx_ref[...] * inv_n
    ex2 = sxx_ref[...] * inv_n
    m2 = mean * a_ref[...]
    var = ex2 - 2.0 * m2 * mean + m2 * m2
    std = jnp.sqrt(var + eps)
    scale = w_ref[...] / std
    off = b_ref[...] - m2 * scale
    o_ref[...] = x_ref[...] * scale + off


def _graph_norm(x, weight, bias, mean_scale, eps=1e-5):
    n, c = x.shape
    blk = 2000 if n % 2000 == 0 else 625
    sx, sxx = pl.pallas_call(
        _gn_stats_body,
        grid=(n // blk,),
        in_specs=[pl.BlockSpec((blk, c), lambda i: (i, 0))],
        out_specs=[pl.BlockSpec((1, c), lambda i: (0, 0))] * 2,
        out_shape=[jax.ShapeDtypeStruct((1, c), jnp.float32)] * 2,
    )(x)
    w = weight.reshape(1, c)
    b = bias.reshape(1, c)
    a = mean_scale.reshape(1, c)
    full = pl.BlockSpec((1, c), lambda i: (0, 0))
    return pl.pallas_call(
        functools.partial(_gn_apply_body, float(n), eps),
        grid=(n // blk,),
        in_specs=[pl.BlockSpec((blk, c), lambda i: (i, 0)),
                  full, full, full, full, full],
        out_specs=pl.BlockSpec((blk, c), lambda i: (i, 0)),
        out_shape=jax.ShapeDtypeStruct((n, c), jnp.float32),
    )(x, sx, sxx, w, b, a)


def kernel(x_0, x_1, x_2, L0_rows, L0_cols, L0_vals, L1d_rows, L1d_cols,
           L1d_vals, L1u_rows, L1u_cols, L1u_vals, L2d_rows, L2d_cols,
           L2d_vals, L2u_rows, L2u_cols, L2u_vals, B1_rows, B1_cols, B1_vals,
           B2_rows, B2_cols, B2_vals, batch, batch_1, W0, W01, W1d, W1u, W10,
           W12, W2d, W2u, W21, Wa1, Wa2, gn1_w, gn1_b, gn1_a, gn2_w, gn2_b,
           gn2_a):
    p0, p10 = _matmuls(x_0, [W0, W10])
    p1d, p1u, p21 = _matmuls(x_1, [W1d, W1u, W21])
    p2d, p2u = _matmuls(x_2, [W2d, W2u])
    x2_out = _sc_spmm(N2, 10240, [
        (L2d_rows, L2d_cols, L2d_vals, p2d),
        (L2u_rows, L2u_cols, L2u_vals, p2u),
        (B2_cols, B2_rows, B2_vals, p21),
    ])
    p12a1 = _matmul2(x_2, W12, x2_out, Wa1)
    y1 = _sc_spmm(N1, 10240, [
        (L1d_rows, L1d_cols, L1d_vals, p1d),
        (L1u_rows, L1u_cols, L1u_vals, p1u),
        (B1_cols, B1_rows, B1_vals, p10),
        (B2_rows, B2_cols, B2_vals, p12a1),
    ])
    x1_out = _graph_norm(y1, gn1_w, gn1_b, gn1_a)
    p01a2 = _matmul2(x_1, W01, x1_out, Wa2)
    y0 = _sc_spmm(N0, 5120, [
        (L0_rows, L0_cols, L0_vals, p0),
        (B1_rows, B1_cols, B1_vals, p01a2),
    ])
    x0_out = _graph_norm(y0, gn2_w, gn2_b, gn2_a)
    return (x0_out, x1_out, x2_out)
